# Initial kernel scaffold; baseline (speedup 1.0000x reference)
import jax
import jax.numpy as jnp
from jax.experimental import pallas as pl


def kernel(h_V, h_E, edge_idx, W1, b1, W2, b2, W3, b3, A, W_in, b_in, W_out, b_out, g0, bt0, g1, bt1):
    raise NotImplementedError("write your pallas kernel here")


# speedup vs baseline: 5.9491x; 5.9491x over previous
import functools

import jax
import jax.numpy as jnp
from jax import lax
from jax.experimental import pallas as pl
from jax.experimental.pallas import tpu as pltpu
from jax.experimental.pallas import tpu_sc as plsc

N = 10000
E = 320000
D = 128
DFF = 512
SCALE = 30.0
EPS = 1e-6

NC = 2
NS = 16
L = 16
NW = NC * NS
EPW = E // NW
CE = 400
NCHUNK = EPW // CE
CED = 200
NCHUNKD = EPW // CED
NP = 10240
RPT = NP // NS
RZB = 128

BE = 3200
RB = BE // 128
NBE = E // BE

BN = 2048

_slope = 0.01


def _leaky(x):
    return jnp.where(x >= 0, x, _slope * x)


def _unpack(p3, n):
    return jnp.swapaxes(p3.reshape(n // 128, 1, 128), 1, 2).reshape(n, 1)


def _pack(col, n):
    return jnp.swapaxes(col.reshape(n // 128, 128, 1), 1, 2).reshape(1, n // 128, 128)


def _node_pre_body(hv_ref, w1a_ref, b1_ref, aa_ref, p_ref, q_ref):
    x = hv_ref[...]
    p_ref[...] = jnp.dot(x, w1a_ref[...], preferred_element_type=jnp.float32) + b1_ref[...]
    q_ref[...] = jnp.dot(x, aa_ref[...], preferred_element_type=jnp.float32)


def _node_pre(h_V, W1a, b1r, Aa):
    BA = 2000
    return pl.pallas_call(
        _node_pre_body,
        grid=(N // BA,),
        in_specs=[
            pl.BlockSpec((BA, D), lambda i: (i, 0)),
            pl.BlockSpec((D, D), lambda i: (0, 0)),
            pl.BlockSpec((1, D), lambda i: (0, 0)),
            pl.BlockSpec((D, 1), lambda i: (0, 0)),
        ],
        out_specs=[
            pl.BlockSpec((BA, D), lambda i: (i, 0)),
            pl.BlockSpec((BA, 1), lambda i: (i, 0)),
        ],
        out_shape=[
            jax.ShapeDtypeStruct((N, D), jnp.float32),
            jax.ShapeDtypeStruct((N, 1), jnp.float32),
        ],
    )(h_V, W1a, b1r, Aa)


def _gather_body(p_hbm, q_hbm, src_hbm, g_out, qe_out,
                 idx_v, rows_v, q_v, qe_v, sem):
    wid = lax.axis_index("s") * NC + lax.axis_index("c")
    base = wid * EPW
    pltpu.sync_copy(q_hbm, q_v)

    def chunk(i, carry):
        off = base + i * CE
        pltpu.sync_copy(src_hbm.at[pl.ds(off, CE)], idx_v)
        pltpu.async_copy(p_hbm.at[idx_v], rows_v, sem).wait()
        pltpu.sync_copy(rows_v, g_out.at[pl.ds(off, CE)])

        def qloop(j, c):
            iv = idx_v[pl.ds(j * L, L)]
            qe_v[pl.ds(j * L, L)] = plsc.load_gather(q_v, [iv])
            return c

        lax.fori_loop(0, CE // L, qloop, 0)
        pltpu.sync_copy(qe_v, qe_out.at[pl.ds(off, CE)])
        return carry

    lax.fori_loop(0, NCHUNK, chunk, 0)


def _gather(P, q1, src):
    mesh = plsc.VectorSubcoreMesh(
        core_axis_name="c", subcore_axis_name="s", num_cores=NC, num_subcores=NS)
    f = functools.partial(
        pl.kernel,
        out_type=[
            jax.ShapeDtypeStruct((E, D), jnp.float32),
            jax.ShapeDtypeStruct((E,), jnp.float32),
        ],
        mesh=mesh,
        scratch_types=[
            pltpu.VMEM((CE,), jnp.int32),
            pltpu.VMEM((CE, D), jnp.float32),
            pltpu.VMEM((N,), jnp.float32),
            pltpu.VMEM((CE,), jnp.float32),
            pltpu.SemaphoreType.DMA,
        ],
        compiler_params=pltpu.CompilerParams(needs_layout_passes=False),
    )(_gather_body)
    return f(P, q1, src)


def _edge_mlp_body(g_ref, he_ref, qep_ref, w1b_ref, w2_ref, b2_ref,
                   w3_ref, b3_ref, ab_ref, wm_ref, attp_ref):
    x = he_ref[...]
    h1 = _leaky(g_ref[...] + jnp.dot(x, w1b_ref[...], preferred_element_type=jnp.float32))
    h2 = _leaky(jnp.dot(h1, w2_ref[...], preferred_element_type=jnp.float32) + b2_ref[...])
    hm = jnp.dot(h2, w3_ref[...], preferred_element_type=jnp.float32) + b3_ref[...]
    qe = _unpack(qep_ref[...], BE)
    logit = qe + jnp.dot(x, ab_ref[...], preferred_element_type=jnp.float32)
    att = jnp.exp(jax.nn.sigmoid(_leaky(logit)))
    wm_ref[...] = hm * att
    attp_ref[...] = _pack(att, BE)


def _edge_mlp(G, h_E, qeP, W1b, W2, b2r, W3, b3r, Ab):
    return pl.pallas_call(
        _edge_mlp_body,
        grid=(NBE,),
        in_specs=[
            pl.BlockSpec((BE, D), lambda i: (i, 0)),
            pl.BlockSpec((BE, D), lambda i: (i, 0)),
            pl.BlockSpec((1, RB, 128), lambda i: (i, 0, 0)),
            pl.BlockSpec((D, D), lambda i: (0, 0)),
            pl.BlockSpec((D, D), lambda i: (0, 0)),
            pl.BlockSpec((1, D), lambda i: (0, 0)),
            pl.BlockSpec((D, D), lambda i: (0, 0)),
            pl.BlockSpec((1, D), lambda i: (0, 0)),
            pl.BlockSpec((D, 1), lambda i: (0, 0)),
        ],
        out_specs=[
            pl.BlockSpec((BE, D), lambda i: (i, 0)),
            pl.BlockSpec((1, RB, 128), lambda i: (i, 0, 0)),
        ],
        out_shape=[
            jax.ShapeDtypeStruct((E, D), jnp.float32),
            jax.ShapeDtypeStruct((NBE, RB, 128), jnp.float32),
        ],
    )(G, h_E, qeP, W1b, W2, b2r, W3, b3r, Ab)


def _scatter_body(wm_hbm, att_hbm, src_hbm, acc_out, asum_out,
                  acc_sh, asum_sh, idx_v, rows_v, a_v, zA, zB, sem):
    cid = lax.axis_index("c")
    sid = lax.axis_index("s")
    wid = sid * NC + cid
    base = wid * EPW

    def zloopA(t, c):
        zA[t // 8, pl.ds((t % 8) * L, L)] = jnp.zeros((L,), jnp.float32)
        return c

    lax.fori_loop(0, RZB * 8, zloopA, 0, unroll=8)

    def zloopB(t, c):
        zB[pl.ds(t * L, L)] = jnp.zeros((L,), jnp.float32)
        return c

    lax.fori_loop(0, RPT // L, zloopB, 0, unroll=8)

    for j in range(RPT // RZB):
        pltpu.sync_copy(zA, acc_sh.at[pl.ds(sid * RPT + j * RZB, RZB)])
    pltpu.sync_copy(zB, asum_sh.at[pl.ds(sid * RPT, RPT)])
    plsc.subcore_barrier()

    def chunk(i, carry):
        off = base + i * CED
        pltpu.sync_copy(src_hbm.at[pl.ds(off, CED)], idx_v)
        pltpu.sync_copy(wm_hbm.at[pl.ds(off, CED)], rows_v)
        pltpu.sync_copy(att_hbm.at[pl.ds(off, CED)], a_v)
        pltpu.sync_copy(rows_v, acc_sh.at[idx_v], add=True)
        pltpu.sync_copy(a_v, asum_sh.at[idx_v], add=True)
        return carry

    lax.fori_loop(0, NCHUNKD, chunk, 0)
    plsc.subcore_barrier()

    for j in range(RPT // RZB):
        r0 = sid * RPT + j * RZB
        pltpu.sync_copy(acc_sh.at[pl.ds(r0, RZB)], acc_out.at[cid, pl.ds(r0, RZB)])
    pltpu.sync_copy(asum_sh.at[pl.ds(sid * RPT, RPT)],
                    asum_out.at[cid, pl.ds(sid * RPT, RPT)])


def _scatter(wm, att1, src):
    mesh = plsc.VectorSubcoreMesh(
        core_axis_name="c", subcore_axis_name="s", num_cores=NC, num_subcores=NS)
    f = functools.partial(
        pl.kernel,
        out_type=[
            jax.ShapeDtypeStruct((NC, NP, D), jnp.float32),
            jax.ShapeDtypeStruct((NC, NP), jnp.float32),
        ],
        mesh=mesh,
        scratch_types=[
            pltpu.VMEM_SHARED((NP, D), jnp.float32),
            pltpu.VMEM_SHARED((NP,), jnp.float32),
            pltpu.VMEM((CED,), jnp.int32),
            pltpu.VMEM((CED, D), jnp.float32),
            pltpu.VMEM((CED,), jnp.float32),
            pltpu.VMEM((RZB, D), jnp.float32),
            pltpu.VMEM((RPT,), jnp.float32),
            pltpu.SemaphoreType.DMA,
        ],
    )(_scatter_body)
    return f(wm, att1, src)


def _node_post_body(hv_ref, acc_ref, asum_ref, win_ref, bin_ref, wout_ref,
                    bout_ref, g0_ref, bt0_ref, g1_ref, bt1_ref, out_ref):
    acc = acc_ref[0] + acc_ref[1]
    asum = _unpack((asum_ref[0] + asum_ref[1]).reshape(1, BN // 128, 128), BN)
    dh = jnp.where(asum > 0, acc / jnp.where(asum > 0, asum, 1.0), 0.0) / SCALE
    x = hv_ref[...] + dh
    mu = jnp.mean(x, axis=-1, keepdims=True)
    xc = x - mu
    var = jnp.mean(xc * xc, axis=-1, keepdims=True)
    x = g0_ref[...] * xc * jax.lax.rsqrt(var + EPS) + bt0_ref[...]
    h = jnp.maximum(jnp.dot(x, win_ref[...], preferred_element_type=jnp.float32) + bin_ref[...], 0.0)
    ffn = jnp.dot(h, wout_ref[...], preferred_element_type=jnp.float32) + bout_ref[...]
    y = x + ffn
    mu2 = jnp.mean(y, axis=-1, keepdims=True)
    yc = y - mu2
    var2 = jnp.mean(yc * yc, axis=-1, keepdims=True)
    out_ref[...] = g1_ref[...] * yc * jax.lax.rsqrt(var2 + EPS) + bt1_ref[...]


def _node_post(h_Vp, accP, asumP3, W_in, binr, W_out, boutr, g0r, bt0r, g1r, bt1r):
    return pl.pallas_call(
        _node_post_body,
        grid=(NP // BN,),
        in_specs=[
            pl.BlockSpec((BN, D), lambda i: (i, 0)),
            pl.BlockSpec((NC, BN, D), lambda i: (0, i, 0)),
            pl.BlockSpec((NC, BN // 128, 128), lambda i: (0, i, 0)),
            pl.BlockSpec((D, DFF), lambda i: (0, 0)),
            pl.BlockSpec((1, DFF), lambda i: (0, 0)),
            pl.BlockSpec((DFF, D), lambda i: (0, 0)),
            pl.BlockSpec((1, D), lambda i: (0, 0)),
            pl.BlockSpec((1, D), lambda i: (0, 0)),
            pl.BlockSpec((1, D), lambda i: (0, 0)),
            pl.BlockSpec((1, D), lambda i: (0, 0)),
            pl.BlockSpec((1, D), lambda i: (0, 0)),
        ],
        out_specs=pl.BlockSpec((BN, D), lambda i: (i, 0)),
        out_shape=jax.ShapeDtypeStruct((NP, D), jnp.float32),
    )(h_Vp, accP, asumP3, W_in, binr, W_out, boutr, g0r, bt0r, g1r, bt1r)


def kernel(h_V, h_E, edge_idx, W1, b1, W2, b2, W3, b3, A,
           W_in, b_in, W_out, b_out, g0, bt0, g1, bt1):
    src = edge_idx[0]
    W1a, W1b = W1[:D], W1[D:]
    Aa, Ab = A[:D], A[D:]
    P, q = _node_pre(h_V, W1a, b1.reshape(1, D), Aa)
    G, qe = _gather(P, q.reshape(N), src)
    wm, attP3 = _edge_mlp(G, h_E, qe.reshape(NBE, RB, 128), W1b, W2,
                          b2.reshape(1, D), W3, b3.reshape(1, D), Ab)
    accP, asumP = _scatter(wm, attP3.reshape(E), src)
    h_Vp = jnp.pad(h_V, ((0, NP - N), (0, 0)))
    out = _node_post(h_Vp, accP, asumP.reshape(NC, NP // 128, 128),
                     W_in, b_in.reshape(1, DFF), W_out, b_out.reshape(1, D),
                     g0.reshape(1, D), bt0.reshape(1, D),
                     g1.reshape(1, D), bt1.reshape(1, D))
    return out[:N]


# speedup vs baseline: 7.2605x; 1.2204x over previous
import functools

import jax
import jax.numpy as jnp
from jax import lax
from jax.experimental import pallas as pl
from jax.experimental.pallas import tpu as pltpu
from jax.experimental.pallas import tpu_sc as plsc

N = 10000
E = 320000
D = 128
DFF = 512
SCALE = 30.0
EPS = 1e-6

NC = 2
NS = 16
L = 16
NW = NC * NS
EPW = E // NW
CE = 200
NCHUNK = EPW // CE
CED = 160
NCHUNKD = 62
CET = EPW - NCHUNKD * CED
NP = 10240
RPT = NP // NS
RZB = 128

BE = 3200
RB = BE // 128
NBE = E // BE

BN = 2048

_slope = 0.01


def _leaky(x):
    return jnp.where(x >= 0, x, _slope * x)


def _unpack(p3, n):
    return jnp.swapaxes(p3.reshape(n // 128, 1, 128), 1, 2).reshape(n, 1)


def _pack(col, n):
    return jnp.swapaxes(col.reshape(n // 128, 128, 1), 1, 2).reshape(1, n // 128, 128)


def _node_pre_body(hv_ref, w1a_ref, b1_ref, aa_ref, p_ref, q_ref):
    x = hv_ref[...]
    p_ref[...] = jnp.dot(x, w1a_ref[...], preferred_element_type=jnp.float32) + b1_ref[...]
    q_ref[...] = jnp.dot(x, aa_ref[...], preferred_element_type=jnp.float32)


def _node_pre(h_V, W1a, b1r, Aa):
    BA = 2000
    return pl.pallas_call(
        _node_pre_body,
        grid=(N // BA,),
        in_specs=[
            pl.BlockSpec((BA, D), lambda i: (i, 0)),
            pl.BlockSpec((D, D), lambda i: (0, 0)),
            pl.BlockSpec((1, D), lambda i: (0, 0)),
            pl.BlockSpec((D, 1), lambda i: (0, 0)),
        ],
        out_specs=[
            pl.BlockSpec((BA, D), lambda i: (i, 0)),
            pl.BlockSpec((BA, 1), lambda i: (i, 0)),
        ],
        out_shape=[
            jax.ShapeDtypeStruct((N, D), jnp.float32),
            jax.ShapeDtypeStruct((N, 1), jnp.float32),
        ],
    )(h_V, W1a, b1r, Aa)


def _gather_body(p_hbm, q_hbm, src_hbm, g_out, qe_out,
                 idx0, idx1, rows0, rows1, q_v, qe0, qe1,
                 gs0, gs1, os0, os1, qs0, qs1):
    idx = (idx0, idx1)
    rows = (rows0, rows1)
    qe = (qe0, qe1)
    gs = (gs0, gs1)
    osm = (os0, os1)
    qs = (qs0, qs1)
    wid = lax.axis_index("s") * NC + lax.axis_index("c")
    base = wid * EPW
    pltpu.sync_copy(q_hbm, q_v)

    pltpu.sync_copy(src_hbm.at[pl.ds(base, CE)], idx[0])
    pltpu.async_copy(p_hbm.at[idx[0]], rows[0], gs[0])

    def pair(i2, carry):
        for b in (0, 1):
            i = i2 * 2 + b
            nb = 1 - b
            off = base + i * CE

            @pl.when(i >= 1)
            def _():
                pltpu.make_async_copy(
                    rows[nb], g_out.at[pl.ds(base, CE)], osm[nb]).wait()

            @pl.when(i + 1 < NCHUNK)
            def _():
                off_n = base + (i + 1) * CE
                pltpu.sync_copy(src_hbm.at[pl.ds(off_n, CE)], idx[nb])
                pltpu.async_copy(p_hbm.at[idx[nb]], rows[nb], gs[nb])

            pltpu.make_async_copy(
                p_hbm.at[pl.ds(base, CE)], rows[b], gs[b]).wait()

            @pl.when(i >= 2)
            def _():
                pltpu.make_async_copy(
                    qe[b], qe_out.at[pl.ds(base, CE)], qs[b]).wait()

            def qloop(j, c):
                iv = idx[b][pl.ds(j * L, L)]
                qe[b][pl.ds(j * L, L)] = plsc.load_gather(q_v, [iv])
                return c

            lax.fori_loop(0, CE // L, qloop, 0)
            ivt = idx[b][pl.ds(CE - L, L)]
            qe[b][pl.ds(CE - L, L)] = plsc.load_gather(q_v, [ivt])

            pltpu.async_copy(qe[b], qe_out.at[pl.ds(off, CE)], qs[b])
            pltpu.async_copy(rows[b], g_out.at[pl.ds(off, CE)], osm[b])
        return carry

    lax.fori_loop(0, NCHUNK // 2, pair, 0)
    pltpu.make_async_copy(rows[1], g_out.at[pl.ds(base, CE)], osm[1]).wait()
    for b in (0, 1):
        pltpu.make_async_copy(qe[b], qe_out.at[pl.ds(base, CE)], qs[b]).wait()


def _gather(P, q1, src):
    mesh = plsc.VectorSubcoreMesh(
        core_axis_name="c", subcore_axis_name="s", num_cores=NC, num_subcores=NS)
    f = functools.partial(
        pl.kernel,
        out_type=[
            jax.ShapeDtypeStruct((E, D), jnp.float32),
            jax.ShapeDtypeStruct((E,), jnp.float32),
        ],
        mesh=mesh,
        scratch_types=[
            pltpu.VMEM((CE,), jnp.int32),
            pltpu.VMEM((CE,), jnp.int32),
            pltpu.VMEM((CE, D), jnp.float32),
            pltpu.VMEM((CE, D), jnp.float32),
            pltpu.VMEM((N,), jnp.float32),
            pltpu.VMEM((CE,), jnp.float32),
            pltpu.VMEM((CE,), jnp.float32),
            pltpu.SemaphoreType.DMA,
            pltpu.SemaphoreType.DMA,
            pltpu.SemaphoreType.DMA,
            pltpu.SemaphoreType.DMA,
            pltpu.SemaphoreType.DMA,
            pltpu.SemaphoreType.DMA,
        ],
        compiler_params=pltpu.CompilerParams(needs_layout_passes=False),
    )(_gather_body)
    return f(P, q1, src)


def _edge_mlp_body(g_ref, he_ref, qep_ref, w1b_ref, w2_ref, b2_ref,
                   w3_ref, b3_ref, ab_ref, wm_ref, attp_ref):
    x = he_ref[...]
    h1 = _leaky(g_ref[...] + jnp.dot(x, w1b_ref[...], preferred_element_type=jnp.float32))
    h2 = _leaky(jnp.dot(h1, w2_ref[...], preferred_element_type=jnp.float32) + b2_ref[...])
    hm = jnp.dot(h2, w3_ref[...], preferred_element_type=jnp.float32) + b3_ref[...]
    qe = _unpack(qep_ref[...], BE)
    logit = qe + jnp.dot(x, ab_ref[...], preferred_element_type=jnp.float32)
    att = jnp.exp(jax.nn.sigmoid(_leaky(logit)))
    wm_ref[...] = hm * att
    attp_ref[...] = _pack(att, BE)


def _edge_mlp(G, h_E, qeP, W1b, W2, b2r, W3, b3r, Ab):
    return pl.pallas_call(
        _edge_mlp_body,
        grid=(NBE,),
        in_specs=[
            pl.BlockSpec((BE, D), lambda i: (i, 0)),
            pl.BlockSpec((BE, D), lambda i: (i, 0)),
            pl.BlockSpec((1, RB, 128), lambda i: (i, 0, 0)),
            pl.BlockSpec((D, D), lambda i: (0, 0)),
            pl.BlockSpec((D, D), lambda i: (0, 0)),
            pl.BlockSpec((1, D), lambda i: (0, 0)),
            pl.BlockSpec((D, D), lambda i: (0, 0)),
            pl.BlockSpec((1, D), lambda i: (0, 0)),
            pl.BlockSpec((D, 1), lambda i: (0, 0)),
        ],
        out_specs=[
            pl.BlockSpec((BE, D), lambda i: (i, 0)),
            pl.BlockSpec((1, RB, 128), lambda i: (i, 0, 0)),
        ],
        out_shape=[
            jax.ShapeDtypeStruct((E, D), jnp.float32),
            jax.ShapeDtypeStruct((NBE, RB, 128), jnp.float32),
        ],
    )(G, h_E, qeP, W1b, W2, b2r, W3, b3r, Ab)


ZW = 32


def _scatter_body(wm_hbm, att_hbm, src_hbm, acc_out, asum_out,
                  acc_sh, asum_sh, idx0, idx1, rows0, rows1, a0, a1,
                  zbuf, zbufB, is0, is1, zs):
    idx = (idx0, idx1)
    rows = (rows0, rows1)
    av = (a0, a1)
    isem = (is0, is1)
    cid = lax.axis_index("c")
    sid = lax.axis_index("s")
    wid = sid * NC + cid
    base = wid * EPW
    r0 = sid * RPT

    def zloopA(t, c):
        zbuf[t // 8, pl.ds((t % 8) * L, L)] = jnp.zeros((L,), jnp.float32)
        return c

    lax.fori_loop(0, ZW * 8, zloopA, 0, unroll=8)

    def zloopB(t, c):
        zbufB[pl.ds(t * L, L)] = jnp.zeros((L,), jnp.float32)
        return c

    lax.fori_loop(0, RPT // L, zloopB, 0, unroll=8)

    for k in range(RPT // ZW):
        pltpu.async_copy(zbuf, acc_sh.at[pl.ds(r0 + k * ZW, ZW)], zs)
    pltpu.async_copy(zbufB, asum_sh.at[pl.ds(r0, RPT)], zs)
    for k in range(RPT // ZW):
        pltpu.make_async_copy(zbuf, acc_sh.at[pl.ds(r0, ZW)], zs).wait()
    pltpu.make_async_copy(zbufB, asum_sh.at[pl.ds(r0, RPT)], zs).wait()
    plsc.subcore_barrier()

    def _issue(i, b):
        off = base + i * CED
        pltpu.async_copy(src_hbm.at[pl.ds(off, CED)], idx[b], isem[b])
        pltpu.async_copy(wm_hbm.at[pl.ds(off, CED)], rows[b], isem[b])
        pltpu.async_copy(att_hbm.at[pl.ds(off, CED)], av[b], isem[b])

    def _drain(b):
        pltpu.make_async_copy(src_hbm.at[pl.ds(base, CED)], idx[b], isem[b]).wait()
        pltpu.make_async_copy(wm_hbm.at[pl.ds(base, CED)], rows[b], isem[b]).wait()
        pltpu.make_async_copy(att_hbm.at[pl.ds(base, CED)], av[b], isem[b]).wait()

    _issue(0, 0)

    def pair(i2, carry):
        for b in (0, 1):
            i = i2 * 2 + b
            nb = 1 - b

            @pl.when(i + 1 < NCHUNKD)
            def _():
                _issue(i + 1, nb)

            _drain(b)
            pltpu.sync_copy(rows[b], acc_sh.at[idx[b]], add=True)
            pltpu.sync_copy(av[b], asum_sh.at[idx[b]], add=True)
        return carry

    lax.fori_loop(0, NCHUNKD // 2, pair, 0)

    offt = base + NCHUNKD * CED
    pltpu.sync_copy(src_hbm.at[pl.ds(offt, CET)], idx[0].at[pl.ds(0, CET)])
    pltpu.sync_copy(wm_hbm.at[pl.ds(offt, CET)], rows[0].at[pl.ds(0, CET)])
    pltpu.sync_copy(att_hbm.at[pl.ds(offt, CET)], av[0].at[pl.ds(0, CET)])
    pltpu.sync_copy(rows[0].at[pl.ds(0, CET)],
                    acc_sh.at[idx[0].at[pl.ds(0, CET)]], add=True)
    pltpu.sync_copy(av[0].at[pl.ds(0, CET)],
                    asum_sh.at[idx[0].at[pl.ds(0, CET)]], add=True)
    plsc.subcore_barrier()

    for j in range(RPT // RZB):
        rr = sid * RPT + j * RZB
        pltpu.sync_copy(acc_sh.at[pl.ds(rr, RZB)], acc_out.at[cid, pl.ds(rr, RZB)])
    pltpu.sync_copy(asum_sh.at[pl.ds(sid * RPT, RPT)],
                    asum_out.at[cid, pl.ds(sid * RPT, RPT)])


def _scatter(wm, att1, src):
    mesh = plsc.VectorSubcoreMesh(
        core_axis_name="c", subcore_axis_name="s", num_cores=NC, num_subcores=NS)
    f = functools.partial(
        pl.kernel,
        out_type=[
            jax.ShapeDtypeStruct((NC, NP, D), jnp.float32),
            jax.ShapeDtypeStruct((NC, NP), jnp.float32),
        ],
        mesh=mesh,
        scratch_types=[
            pltpu.VMEM_SHARED((NP, D), jnp.float32),
            pltpu.VMEM_SHARED((NP,), jnp.float32),
            pltpu.VMEM((CED,), jnp.int32),
            pltpu.VMEM((CED,), jnp.int32),
            pltpu.VMEM((CED, D), jnp.float32),
            pltpu.VMEM((CED, D), jnp.float32),
            pltpu.VMEM((CED,), jnp.float32),
            pltpu.VMEM((CED,), jnp.float32),
            pltpu.VMEM((ZW, D), jnp.float32),
            pltpu.VMEM((RPT,), jnp.float32),
            pltpu.SemaphoreType.DMA,
            pltpu.SemaphoreType.DMA,
            pltpu.SemaphoreType.DMA,
        ],
    )(_scatter_body)
    return f(wm, att1, src)


def _node_post_body(hv_ref, acc_ref, asum_ref, win_ref, bin_ref, wout_ref,
                    bout_ref, g0_ref, bt0_ref, g1_ref, bt1_ref, out_ref):
    acc = acc_ref[0] + acc_ref[1]
    asum = _unpack((asum_ref[0] + asum_ref[1]).reshape(1, BN // 128, 128), BN)
    dh = jnp.where(asum > 0, acc / jnp.where(asum > 0, asum, 1.0), 0.0) / SCALE
    x = hv_ref[...] + dh
    mu = jnp.mean(x, axis=-1, keepdims=True)
    xc = x - mu
    var = jnp.mean(xc * xc, axis=-1, keepdims=True)
    x = g0_ref[...] * xc * jax.lax.rsqrt(var + EPS) + bt0_ref[...]
    h = jnp.maximum(jnp.dot(x, win_ref[...], preferred_element_type=jnp.float32) + bin_ref[...], 0.0)
    ffn = jnp.dot(h, wout_ref[...], preferred_element_type=jnp.float32) + bout_ref[...]
    y = x + ffn
    mu2 = jnp.mean(y, axis=-1, keepdims=True)
    yc = y - mu2
    var2 = jnp.mean(yc * yc, axis=-1, keepdims=True)
    out_ref[...] = g1_ref[...] * yc * jax.lax.rsqrt(var2 + EPS) + bt1_ref[...]


def _node_post(h_Vp, accP, asumP3, W_in, binr, W_out, boutr, g0r, bt0r, g1r, bt1r):
    return pl.pallas_call(
        _node_post_body,
        grid=(NP // BN,),
        in_specs=[
            pl.BlockSpec((BN, D), lambda i: (i, 0)),
            pl.BlockSpec((NC, BN, D), lambda i: (0, i, 0)),
            pl.BlockSpec((NC, BN // 128, 128), lambda i: (0, i, 0)),
            pl.BlockSpec((D, DFF), lambda i: (0, 0)),
            pl.BlockSpec((1, DFF), lambda i: (0, 0)),
            pl.BlockSpec((DFF, D), lambda i: (0, 0)),
            pl.BlockSpec((1, D), lambda i: (0, 0)),
            pl.BlockSpec((1, D), lambda i: (0, 0)),
            pl.BlockSpec((1, D), lambda i: (0, 0)),
            pl.BlockSpec((1, D), lambda i: (0, 0)),
            pl.BlockSpec((1, D), lambda i: (0, 0)),
        ],
        out_specs=pl.BlockSpec((BN, D), lambda i: (i, 0)),
        out_shape=jax.ShapeDtypeStruct((NP, D), jnp.float32),
    )(h_Vp, accP, asumP3, W_in, binr, W_out, boutr, g0r, bt0r, g1r, bt1r)


def kernel(h_V, h_E, edge_idx, W1, b1, W2, b2, W3, b3, A,
           W_in, b_in, W_out, b_out, g0, bt0, g1, bt1):
    src = edge_idx[0]
    W1a, W1b = W1[:D], W1[D:]
    Aa, Ab = A[:D], A[D:]
    P, q = _node_pre(h_V, W1a, b1.reshape(1, D), Aa)
    G, qe = _gather(P, q.reshape(N), src)
    wm, attP3 = _edge_mlp(G, h_E, qe.reshape(NBE, RB, 128), W1b, W2,
                          b2.reshape(1, D), W3, b3.reshape(1, D), Ab)
    accP, asumP = _scatter(wm, attP3.reshape(E), src)
    h_Vp = jnp.pad(h_V, ((0, NP - N), (0, 0)))
    out = _node_post(h_Vp, accP, asumP.reshape(NC, NP // 128, 128),
                     W_in, b_in.reshape(1, DFF), W_out, b_out.reshape(1, D),
                     g0.reshape(1, D), bt0.reshape(1, D),
                     g1.reshape(1, D), bt1.reshape(1, D))
    return out[:N]


# speedup vs baseline: 7.9033x; 1.0885x over previous
import functools

import jax
import jax.numpy as jnp
from jax import lax
from jax.experimental import pallas as pl
from jax.experimental.pallas import tpu as pltpu
from jax.experimental.pallas import tpu_sc as plsc

N = 10000
E = 320000
D = 128
DFF = 512
SCALE = 30.0
EPS = 1e-6

NC = 2
NS = 16
L = 16
NW = NC * NS

NH = 2
E2 = E // NH
EPW = E2 // NW

CE = 200
NCHUNK = EPW // CE
CED = 160
NCHUNKD = 31
CDT = EPW - NCHUNKD * CED

NP = 10240
RPT = NP // NS
RZB = 128
ZW = 32

BE = 3200
RB = BE // 128
NBE = E2 // BE

BN = 2048

_slope = 0.01


def _leaky(x):
    return jnp.where(x >= 0, x, _slope * x)


def _unpack(p3, n):
    return jnp.swapaxes(p3.reshape(n // 128, 1, 128), 1, 2).reshape(n, 1)


def _pack(col, n):
    return jnp.swapaxes(col.reshape(n // 128, 128, 1), 1, 2).reshape(1, n // 128, 128)


def _node_pre_body(hv_ref, w1a_ref, b1_ref, aa_ref, p_ref, q_ref):
    x = hv_ref[...]
    p_ref[...] = jnp.dot(x, w1a_ref[...], preferred_element_type=jnp.float32) + b1_ref[...]
    q_ref[...] = jnp.dot(x, aa_ref[...], preferred_element_type=jnp.float32)


def _node_pre(h_V, W1a, b1r, Aa):
    BA = 2000
    return pl.pallas_call(
        _node_pre_body,
        grid=(N // BA,),
        in_specs=[
            pl.BlockSpec((BA, D), lambda i: (i, 0)),
            pl.BlockSpec((D, D), lambda i: (0, 0)),
            pl.BlockSpec((1, D), lambda i: (0, 0)),
            pl.BlockSpec((D, 1), lambda i: (0, 0)),
        ],
        out_specs=[
            pl.BlockSpec((BA, D), lambda i: (i, 0)),
            pl.BlockSpec((BA, 1), lambda i: (i, 0)),
        ],
        out_shape=[
            jax.ShapeDtypeStruct((N, D), jnp.float32),
            jax.ShapeDtypeStruct((N, 1), jnp.float32),
        ],
    )(h_V, W1a, b1r, Aa)


def _make_gather_body(h):
    def body(p_hbm, q_hbm, src_hbm, g_out, qe_out,
             idx0, idx1, rows0, rows1, q_v, qe0, qe1,
             gs0, gs1, os0, os1, qs0, qs1):
        idx = (idx0, idx1)
        rows = (rows0, rows1)
        qe = (qe0, qe1)
        gs = (gs0, gs1)
        osm = (os0, os1)
        qs = (qs0, qs1)
        wid = lax.axis_index("s") * NC + lax.axis_index("c")
        bl = wid * EPW
        bg = bl + h * E2
        pltpu.sync_copy(q_hbm, q_v)

        pltpu.sync_copy(src_hbm.at[pl.ds(bg, CE)], idx[0])
        pltpu.async_copy(p_hbm.at[idx[0]], rows[0], gs[0])

        def step(i, b):
            nb = 1 - b
            off = bl + i * CE

            @pl.when(i >= 1)
            def _():
                pltpu.make_async_copy(
                    rows[nb], g_out.at[pl.ds(bl, CE)], osm[nb]).wait()

            @pl.when(i + 1 < NCHUNK)
            def _():
                off_n = bg + (i + 1) * CE
                pltpu.sync_copy(src_hbm.at[pl.ds(off_n, CE)], idx[nb])
                pltpu.async_copy(p_hbm.at[idx[nb]], rows[nb], gs[nb])

            pltpu.make_async_copy(
                p_hbm.at[pl.ds(0, CE)], rows[b], gs[b]).wait()

            @pl.when(i >= 2)
            def _():
                pltpu.make_async_copy(
                    qe[b], qe_out.at[pl.ds(bl, CE)], qs[b]).wait()

            def qloop(j, c):
                iv = idx[b][pl.ds(j * L, L)]
                qe[b][pl.ds(j * L, L)] = plsc.load_gather(q_v, [iv])
                return c

            lax.fori_loop(0, CE // L, qloop, 0)
            ivt = idx[b][pl.ds(CE - L, L)]
            qe[b][pl.ds(CE - L, L)] = plsc.load_gather(q_v, [ivt])

            pltpu.async_copy(qe[b], qe_out.at[pl.ds(off, CE)], qs[b])
            pltpu.async_copy(rows[b], g_out.at[pl.ds(off, CE)], osm[b])

        def pair(i2, carry):
            for b in (0, 1):
                step(i2 * 2 + b, b)
            return carry

        lax.fori_loop(0, NCHUNK // 2, pair, 0)
        if NCHUNK % 2:
            step(NCHUNK - 1, 0)
        lb = (NCHUNK - 1) % 2
        pltpu.make_async_copy(rows[lb], g_out.at[pl.ds(bl, CE)], osm[lb]).wait()
        for b in (0, 1):
            pltpu.make_async_copy(qe[b], qe_out.at[pl.ds(bl, CE)], qs[b]).wait()

    return body


def _gather(P, q1, src, h):
    mesh = plsc.VectorSubcoreMesh(
        core_axis_name="c", subcore_axis_name="s", num_cores=NC, num_subcores=NS)
    f = functools.partial(
        pl.kernel,
        out_type=[
            jax.ShapeDtypeStruct((E2, D), jnp.float32),
            jax.ShapeDtypeStruct((E2,), jnp.float32),
        ],
        mesh=mesh,
        scratch_types=[
            pltpu.VMEM((CE,), jnp.int32),
            pltpu.VMEM((CE,), jnp.int32),
            pltpu.VMEM((CE, D), jnp.float32),
            pltpu.VMEM((CE, D), jnp.float32),
            pltpu.VMEM((N,), jnp.float32),
            pltpu.VMEM((CE,), jnp.float32),
            pltpu.VMEM((CE,), jnp.float32),
            pltpu.SemaphoreType.DMA,
            pltpu.SemaphoreType.DMA,
            pltpu.SemaphoreType.DMA,
            pltpu.SemaphoreType.DMA,
            pltpu.SemaphoreType.DMA,
            pltpu.SemaphoreType.DMA,
        ],
        compiler_params=pltpu.CompilerParams(needs_layout_passes=False),
        name=f"edge_gather_h{h}",
    )(_make_gather_body(h))
    return f(P, q1, src)


def _edge_mlp_body(g_ref, he_ref, qep_ref, w1b_ref, w2_ref, b2_ref,
                   w3_ref, b3_ref, ab_ref, wm_ref, attp_ref):
    x = he_ref[...]
    h1 = _leaky(g_ref[...] + jnp.dot(x, w1b_ref[...], preferred_element_type=jnp.float32))
    h2 = _leaky(jnp.dot(h1, w2_ref[...], preferred_element_type=jnp.float32) + b2_ref[...])
    hm = jnp.dot(h2, w3_ref[...], preferred_element_type=jnp.float32) + b3_ref[...]
    qe = _unpack(qep_ref[...], BE)
    logit = qe + jnp.dot(x, ab_ref[...], preferred_element_type=jnp.float32)
    att = jnp.exp(jax.nn.sigmoid(_leaky(logit)))
    wm_ref[...] = hm * att
    attp_ref[...] = _pack(att, BE)


def _edge_mlp(G, h_E, qeP, W1b, W2, b2r, W3, b3r, Ab, h):
    off = h * NBE

    def he_map(i):
        return (i + off, 0)

    return pl.pallas_call(
        _edge_mlp_body,
        grid=(NBE,),
        in_specs=[
            pl.BlockSpec((BE, D), lambda i: (i, 0)),
            pl.BlockSpec((BE, D), he_map),
            pl.BlockSpec((1, RB, 128), lambda i: (i, 0, 0)),
            pl.BlockSpec((D, D), lambda i: (0, 0)),
            pl.BlockSpec((D, D), lambda i: (0, 0)),
            pl.BlockSpec((1, D), lambda i: (0, 0)),
            pl.BlockSpec((D, D), lambda i: (0, 0)),
            pl.BlockSpec((1, D), lambda i: (0, 0)),
            pl.BlockSpec((D, 1), lambda i: (0, 0)),
        ],
        out_specs=[
            pl.BlockSpec((BE, D), lambda i: (i, 0)),
            pl.BlockSpec((1, RB, 128), lambda i: (i, 0, 0)),
        ],
        out_shape=[
            jax.ShapeDtypeStruct((E2, D), jnp.float32),
            jax.ShapeDtypeStruct((NBE, RB, 128), jnp.float32),
        ],
        name=f"edge_mlp_h{h}",
    )(G, h_E, qeP, W1b, W2, b2r, W3, b3r, Ab)


def _make_scatter_body(h):
    def body(wm_hbm, att_hbm, src_hbm, acc_out, asum_out,
             acc_sh, asum_sh, idx0, idx1, rows0, rows1, a0, a1,
             zbuf, zbufB, is0, is1, zs):
        idx = (idx0, idx1)
        rows = (rows0, rows1)
        av = (a0, a1)
        isem = (is0, is1)
        cid = lax.axis_index("c")
        sid = lax.axis_index("s")
        wid = sid * NC + cid
        bl = wid * EPW
        bg = bl + h * E2
        r0 = sid * RPT

        def zloopA(t, c):
            zbuf[t // 8, pl.ds((t % 8) * L, L)] = jnp.zeros((L,), jnp.float32)
            return c

        lax.fori_loop(0, ZW * 8, zloopA, 0, unroll=8)

        def zloopB(t, c):
            zbufB[pl.ds(t * L, L)] = jnp.zeros((L,), jnp.float32)
            return c

        lax.fori_loop(0, RPT // L, zloopB, 0, unroll=8)

        for k in range(RPT // ZW):
            pltpu.async_copy(zbuf, acc_sh.at[pl.ds(r0 + k * ZW, ZW)], zs)
        pltpu.async_copy(zbufB, asum_sh.at[pl.ds(r0, RPT)], zs)
        for k in range(RPT // ZW):
            pltpu.make_async_copy(zbuf, acc_sh.at[pl.ds(r0, ZW)], zs).wait()
        pltpu.make_async_copy(zbufB, asum_sh.at[pl.ds(r0, RPT)], zs).wait()
        plsc.subcore_barrier()

        def _issue(i, b):
            off = bg + i * CED
            pltpu.async_copy(src_hbm.at[pl.ds(off, CED)], idx[b], isem[b])
            offl = bl + i * CED
            pltpu.async_copy(wm_hbm.at[pl.ds(offl, CED)], rows[b], isem[b])
            pltpu.async_copy(att_hbm.at[pl.ds(offl, CED)], av[b], isem[b])

        def _drain(b):
            pltpu.make_async_copy(src_hbm.at[pl.ds(0, CED)], idx[b], isem[b]).wait()
            pltpu.make_async_copy(wm_hbm.at[pl.ds(0, CED)], rows[b], isem[b]).wait()
            pltpu.make_async_copy(att_hbm.at[pl.ds(0, CED)], av[b], isem[b]).wait()

        def step(i, b):
            nb = 1 - b

            @pl.when(i + 1 < NCHUNKD)
            def _():
                _issue(i + 1, nb)

            _drain(b)
            pltpu.sync_copy(rows[b], acc_sh.at[idx[b]], add=True)
            pltpu.sync_copy(av[b], asum_sh.at[idx[b]], add=True)

        _issue(0, 0)

        def pair(i2, carry):
            for b in (0, 1):
                step(i2 * 2 + b, b)
            return carry

        lax.fori_loop(0, NCHUNKD // 2, pair, 0)
        if NCHUNKD % 2:
            step(NCHUNKD - 1, 0)

        offt_g = bg + NCHUNKD * CED
        offt_l = bl + NCHUNKD * CED
        pltpu.sync_copy(src_hbm.at[pl.ds(offt_g, CDT)], idx[0].at[pl.ds(0, CDT)])
        pltpu.sync_copy(wm_hbm.at[pl.ds(offt_l, CDT)], rows[0].at[pl.ds(0, CDT)])
        pltpu.sync_copy(att_hbm.at[pl.ds(offt_l, CDT)], av[0].at[pl.ds(0, CDT)])
        pltpu.sync_copy(rows[0].at[pl.ds(0, CDT)],
                        acc_sh.at[idx[0].at[pl.ds(0, CDT)]], add=True)
        pltpu.sync_copy(av[0].at[pl.ds(0, CDT)],
                        asum_sh.at[idx[0].at[pl.ds(0, CDT)]], add=True)
        plsc.subcore_barrier()

        for j in range(RPT // RZB):
            rr = sid * RPT + j * RZB
            pltpu.sync_copy(acc_sh.at[pl.ds(rr, RZB)], acc_out.at[cid, pl.ds(rr, RZB)])
        pltpu.sync_copy(asum_sh.at[pl.ds(sid * RPT, RPT)],
                        asum_out.at[cid, pl.ds(sid * RPT, RPT)])

    return body


def _scatter(wm, att1, src, h):
    mesh = plsc.VectorSubcoreMesh(
        core_axis_name="c", subcore_axis_name="s", num_cores=NC, num_subcores=NS)
    f = functools.partial(
        pl.kernel,
        out_type=[
            jax.ShapeDtypeStruct((NC, NP, D), jnp.float32),
            jax.ShapeDtypeStruct((NC, NP), jnp.float32),
        ],
        mesh=mesh,
        scratch_types=[
            pltpu.VMEM_SHARED((NP, D), jnp.float32),
            pltpu.VMEM_SHARED((NP,), jnp.float32),
            pltpu.VMEM((CED,), jnp.int32),
            pltpu.VMEM((CED,), jnp.int32),
            pltpu.VMEM((CED, D), jnp.float32),
            pltpu.VMEM((CED, D), jnp.float32),
            pltpu.VMEM((CED,), jnp.float32),
            pltpu.VMEM((CED,), jnp.float32),
            pltpu.VMEM((ZW, D), jnp.float32),
            pltpu.VMEM((RPT,), jnp.float32),
            pltpu.SemaphoreType.DMA,
            pltpu.SemaphoreType.DMA,
            pltpu.SemaphoreType.DMA,
        ],
        name=f"edge_scatter_h{h}",
    )(_make_scatter_body(h))
    return f(wm, att1, src)


def _node_post_body(hv_ref, acc1_ref, acc2_ref, asum1_ref, asum2_ref,
                    win_ref, bin_ref, wout_ref,
                    bout_ref, g0_ref, bt0_ref, g1_ref, bt1_ref, out_ref):
    acc = acc1_ref[0] + acc1_ref[1] + acc2_ref[0] + acc2_ref[1]
    asum_p = (asum1_ref[0] + asum1_ref[1] + asum2_ref[0] + asum2_ref[1])
    asum = _unpack(asum_p.reshape(1, BN // 128, 128), BN)
    dh = jnp.where(asum > 0, acc / jnp.where(asum > 0, asum, 1.0), 0.0) / SCALE
    x = hv_ref[...] + dh
    mu = jnp.mean(x, axis=-1, keepdims=True)
    xc = x - mu
    var = jnp.mean(xc * xc, axis=-1, keepdims=True)
    x = g0_ref[...] * xc * jax.lax.rsqrt(var + EPS) + bt0_ref[...]
    hh = jnp.maximum(jnp.dot(x, win_ref[...], preferred_element_type=jnp.float32) + bin_ref[...], 0.0)
    ffn = jnp.dot(hh, wout_ref[...], preferred_element_type=jnp.float32) + bout_ref[...]
    y = x + ffn
    mu2 = jnp.mean(y, axis=-1, keepdims=True)
    yc = y - mu2
    var2 = jnp.mean(yc * yc, axis=-1, keepdims=True)
    out_ref[...] = g1_ref[...] * yc * jax.lax.rsqrt(var2 + EPS) + bt1_ref[...]


def _node_post(h_Vp, acc1, acc2, asum1, asum2,
               W_in, binr, W_out, boutr, g0r, bt0r, g1r, bt1r):
    return pl.pallas_call(
        _node_post_body,
        grid=(NP // BN,),
        in_specs=[
            pl.BlockSpec((BN, D), lambda i: (i, 0)),
            pl.BlockSpec((NC, BN, D), lambda i: (0, i, 0)),
            pl.BlockSpec((NC, BN, D), lambda i: (0, i, 0)),
            pl.BlockSpec((NC, BN // 128, 128), lambda i: (0, i, 0)),
            pl.BlockSpec((NC, BN // 128, 128), lambda i: (0, i, 0)),
            pl.BlockSpec((D, DFF), lambda i: (0, 0)),
            pl.BlockSpec((1, DFF), lambda i: (0, 0)),
            pl.BlockSpec((DFF, D), lambda i: (0, 0)),
            pl.BlockSpec((1, D), lambda i: (0, 0)),
            pl.BlockSpec((1, D), lambda i: (0, 0)),
            pl.BlockSpec((1, D), lambda i: (0, 0)),
            pl.BlockSpec((1, D), lambda i: (0, 0)),
            pl.BlockSpec((1, D), lambda i: (0, 0)),
        ],
        out_specs=pl.BlockSpec((BN, D), lambda i: (i, 0)),
        out_shape=jax.ShapeDtypeStruct((NP, D), jnp.float32),
    )(h_Vp, acc1, acc2, asum1, asum2,
      W_in, binr, W_out, boutr, g0r, bt0r, g1r, bt1r)


def kernel(h_V, h_E, edge_idx, W1, b1, W2, b2, W3, b3, A,
           W_in, b_in, W_out, b_out, g0, bt0, g1, bt1):
    src = edge_idx[0]
    W1a, W1b = W1[:D], W1[D:]
    Aa, Ab = A[:D], A[D:]
    P, q = _node_pre(h_V, W1a, b1.reshape(1, D), Aa)
    q1 = q.reshape(N)
    b2r, b3r = b2.reshape(1, D), b3.reshape(1, D)

    G = [None] * NH
    qe = [None] * NH
    wm = [None] * NH
    att = [None] * NH
    acc = [None] * NH
    asum = [None] * NH
    for h in range(NH):
        G[h], qe[h] = _gather(P, q1, src, h)
    for h in range(NH):
        wm[h], attP3 = _edge_mlp(G[h], h_E, qe[h].reshape(NBE, RB, 128),
                                 W1b, W2, b2r, W3, b3r, Ab, h)
        att[h] = attP3.reshape(E2)
    for h in range(NH):
        acc[h], asum[h] = _scatter(wm[h], att[h], src, h)

    h_Vp = jnp.pad(h_V, ((0, NP - N), (0, 0)))
    out = _node_post(h_Vp, acc[0], acc[1],
                     asum[0].reshape(NC, NP // 128, 128),
                     asum[1].reshape(NC, NP // 128, 128),
                     W_in, b_in.reshape(1, DFF), W_out, b_out.reshape(1, D),
                     g0.reshape(1, D), bt0.reshape(1, D),
                     g1.reshape(1, D), bt1.reshape(1, D))
    return out[:N]


# speedup vs baseline: 8.0095x; 1.0134x over previous
import functools

import jax
import jax.numpy as jnp
from jax import lax
from jax.experimental import pallas as pl
from jax.experimental.pallas import tpu as pltpu
from jax.experimental.pallas import tpu_sc as plsc

N = 10000
E = 320000
D = 128
DFF = 512
SCALE = 30.0
EPS = 1e-6

NC = 2
NS = 16
L = 16
NW = NC * NS

NH = 4
E2 = E // NH

CE = 200
NCHG = E2 // CE
KMAXG = (NCHG + NW - 1) // NW
CED = 160
NCHD = E2 // CED
KMAXD = (NCHD + NW - 1) // NW

NP = 10240
RPT = NP // NS
RZB = 128
ZW = 32

BE = 3200
RB = BE // 128
NBE = E2 // BE

BN = 2048

_slope = 0.01


def _leaky(x):
    return jnp.where(x >= 0, x, _slope * x)


def _unpack(p3, n):
    return jnp.swapaxes(p3.reshape(n // 128, 1, 128), 1, 2).reshape(n, 1)


def _pack(col, n):
    return jnp.swapaxes(col.reshape(n // 128, 128, 1), 1, 2).reshape(1, n // 128, 128)


def _node_pre_body(hv_ref, w1a_ref, b1_ref, aa_ref, p_ref, q_ref):
    x = hv_ref[...]
    p_ref[...] = jnp.dot(x, w1a_ref[...], preferred_element_type=jnp.float32) + b1_ref[...]
    q_ref[...] = jnp.dot(x, aa_ref[...], preferred_element_type=jnp.float32)


def _node_pre(h_V, W1a, b1r, Aa):
    BA = 2000
    return pl.pallas_call(
        _node_pre_body,
        grid=(N // BA,),
        in_specs=[
            pl.BlockSpec((BA, D), lambda i: (i, 0)),
            pl.BlockSpec((D, D), lambda i: (0, 0)),
            pl.BlockSpec((1, D), lambda i: (0, 0)),
            pl.BlockSpec((D, 1), lambda i: (0, 0)),
        ],
        out_specs=[
            pl.BlockSpec((BA, D), lambda i: (i, 0)),
            pl.BlockSpec((BA, 1), lambda i: (i, 0)),
        ],
        out_shape=[
            jax.ShapeDtypeStruct((N, D), jnp.float32),
            jax.ShapeDtypeStruct((N, 1), jnp.float32),
        ],
    )(h_V, W1a, b1r, Aa)


def _make_gather_body(h):
    def body(p_hbm, q_hbm, src_hbm, g_out, qe_out,
             idx0, idx1, rows0, rows1, q_v, qe0, qe1,
             gs0, gs1, os0, os1, qs0, qs1):
        idx = (idx0, idx1)
        rows = (rows0, rows1)
        qe = (qe0, qe1)
        gs = (gs0, gs1)
        osm = (os0, os1)
        qs = (qs0, qs1)
        wid = lax.axis_index("s") * NC + lax.axis_index("c")
        pltpu.sync_copy(q_hbm, q_v)

        def _issue(c, b):
            pltpu.sync_copy(src_hbm.at[pl.ds(c * CE + h * E2, CE)], idx[b])
            pltpu.async_copy(p_hbm.at[idx[b]], rows[b], gs[b])

        _issue(wid, 0)

        def step(k, b):
            nb = 1 - b
            ck = wid + k * NW
            cn = wid + (k + 1) * NW

            @pl.when(jnp.logical_and(k >= 1, cn < NCHG))
            def _():
                pltpu.make_async_copy(
                    rows[nb], g_out.at[pl.ds(0, CE)], osm[nb]).wait()

            @pl.when(cn < NCHG)
            def _():
                _issue(cn, nb)

            @pl.when(ck < NCHG)
            def _():
                pltpu.make_async_copy(
                    p_hbm.at[pl.ds(0, CE)], rows[b], gs[b]).wait()

                @pl.when(k >= 2)
                def _():
                    pltpu.make_async_copy(
                        qe[b], qe_out.at[pl.ds(0, CE)], qs[b]).wait()

                def qloop(j, c):
                    iv = idx[b][pl.ds(j * L, L)]
                    qe[b][pl.ds(j * L, L)] = plsc.load_gather(q_v, [iv])
                    return c

                lax.fori_loop(0, CE // L, qloop, 0)
                ivt = idx[b][pl.ds(CE - L, L)]
                qe[b][pl.ds(CE - L, L)] = plsc.load_gather(q_v, [ivt])

                off = ck * CE
                pltpu.async_copy(qe[b], qe_out.at[pl.ds(off, CE)], qs[b])
                pltpu.async_copy(rows[b], g_out.at[pl.ds(off, CE)], osm[b])

        def pair(k2, carry):
            for b in (0, 1):
                step(k2 * 2 + b, b)
            return carry

        lax.fori_loop(0, KMAXG // 2, pair, 0)
        if KMAXG % 2:
            step(KMAXG - 1, 0)

        for kk in (KMAXG - 3, KMAXG - 2, KMAXG - 1):
            ckk = wid + kk * NW

            @pl.when(jnp.logical_and(ckk < NCHG, ckk >= NCHG - 2 * NW))
            def _():
                pltpu.make_async_copy(
                    rows[kk % 2], g_out.at[pl.ds(0, CE)], osm[kk % 2]).wait()
                pltpu.make_async_copy(
                    qe[kk % 2], qe_out.at[pl.ds(0, CE)], qs[kk % 2]).wait()

    return body


def _gather(P, q1, src, h):
    mesh = plsc.VectorSubcoreMesh(
        core_axis_name="c", subcore_axis_name="s", num_cores=NC, num_subcores=NS)
    f = functools.partial(
        pl.kernel,
        out_type=[
            jax.ShapeDtypeStruct((E2, D), jnp.float32),
            jax.ShapeDtypeStruct((E2,), jnp.float32),
        ],
        mesh=mesh,
        scratch_types=[
            pltpu.VMEM((CE,), jnp.int32),
            pltpu.VMEM((CE,), jnp.int32),
            pltpu.VMEM((CE, D), jnp.float32),
            pltpu.VMEM((CE, D), jnp.float32),
            pltpu.VMEM((N,), jnp.float32),
            pltpu.VMEM((CE,), jnp.float32),
            pltpu.VMEM((CE,), jnp.float32),
            pltpu.SemaphoreType.DMA,
            pltpu.SemaphoreType.DMA,
            pltpu.SemaphoreType.DMA,
            pltpu.SemaphoreType.DMA,
            pltpu.SemaphoreType.DMA,
            pltpu.SemaphoreType.DMA,
        ],
        compiler_params=pltpu.CompilerParams(needs_layout_passes=False),
        name=f"edge_gather_h{h}",
    )(_make_gather_body(h))
    return f(P, q1, src)


def _edge_mlp_body(g_ref, he_ref, qep_ref, w1b_ref, w2_ref, b2_ref,
                   w3_ref, b3_ref, ab_ref, wm_ref, attp_ref):
    x = he_ref[...]
    h1 = _leaky(g_ref[...] + jnp.dot(x, w1b_ref[...], preferred_element_type=jnp.float32))
    h2 = _leaky(jnp.dot(h1, w2_ref[...], preferred_element_type=jnp.float32) + b2_ref[...])
    hm = jnp.dot(h2, w3_ref[...], preferred_element_type=jnp.float32) + b3_ref[...]
    qe = _unpack(qep_ref[...], BE)
    logit = qe + jnp.dot(x, ab_ref[...], preferred_element_type=jnp.float32)
    att = jnp.exp(jax.nn.sigmoid(_leaky(logit)))
    wm_ref[...] = hm * att
    attp_ref[...] = _pack(att, BE)


def _edge_mlp(G, h_E, qeP, W1b, W2, b2r, W3, b3r, Ab, h):
    off = h * NBE

    def he_map(i):
        return (i + off, 0)

    return pl.pallas_call(
        _edge_mlp_body,
        grid=(NBE,),
        in_specs=[
            pl.BlockSpec((BE, D), lambda i: (i, 0)),
            pl.BlockSpec((BE, D), he_map),
            pl.BlockSpec((1, RB, 128), lambda i: (i, 0, 0)),
            pl.BlockSpec((D, D), lambda i: (0, 0)),
            pl.BlockSpec((D, D), lambda i: (0, 0)),
            pl.BlockSpec((1, D), lambda i: (0, 0)),
            pl.BlockSpec((D, D), lambda i: (0, 0)),
            pl.BlockSpec((1, D), lambda i: (0, 0)),
            pl.BlockSpec((D, 1), lambda i: (0, 0)),
        ],
        out_specs=[
            pl.BlockSpec((BE, D), lambda i: (i, 0)),
            pl.BlockSpec((1, RB, 128), lambda i: (i, 0, 0)),
        ],
        out_shape=[
            jax.ShapeDtypeStruct((E2, D), jnp.float32),
            jax.ShapeDtypeStruct((NBE, RB, 128), jnp.float32),
        ],
        name=f"edge_mlp_h{h}",
    )(G, h_E, qeP, W1b, W2, b2r, W3, b3r, Ab)


def _make_scatter_body(h):
    def body(wm_hbm, att_hbm, src_hbm, acc_out, asum_out,
             acc_sh, asum_sh, idx0, idx1, rows0, rows1, a0, a1,
             zbuf, zbufB, is0, is1, zs):
        idx = (idx0, idx1)
        rows = (rows0, rows1)
        av = (a0, a1)
        isem = (is0, is1)
        cid = lax.axis_index("c")
        sid = lax.axis_index("s")
        wid = sid * NC + cid
        r0 = sid * RPT

        def zloopA(t, c):
            zbuf[t // 8, pl.ds((t % 8) * L, L)] = jnp.zeros((L,), jnp.float32)
            return c

        lax.fori_loop(0, ZW * 8, zloopA, 0, unroll=8)

        def zloopB(t, c):
            zbufB[pl.ds(t * L, L)] = jnp.zeros((L,), jnp.float32)
            return c

        lax.fori_loop(0, RPT // L, zloopB, 0, unroll=8)

        for k in range(RPT // ZW):
            pltpu.async_copy(zbuf, acc_sh.at[pl.ds(r0 + k * ZW, ZW)], zs)
        pltpu.async_copy(zbufB, asum_sh.at[pl.ds(r0, RPT)], zs)
        for k in range(RPT // ZW):
            pltpu.make_async_copy(zbuf, acc_sh.at[pl.ds(r0, ZW)], zs).wait()
        pltpu.make_async_copy(zbufB, asum_sh.at[pl.ds(r0, RPT)], zs).wait()
        plsc.subcore_barrier()

        def _issue(c, b):
            off = c * CED
            pltpu.async_copy(src_hbm.at[pl.ds(off + h * E2, CED)], idx[b], isem[b])
            pltpu.async_copy(wm_hbm.at[pl.ds(off, CED)], rows[b], isem[b])
            pltpu.async_copy(att_hbm.at[pl.ds(off, CED)], av[b], isem[b])

        def _drain(b):
            pltpu.make_async_copy(src_hbm.at[pl.ds(0, CED)], idx[b], isem[b]).wait()
            pltpu.make_async_copy(wm_hbm.at[pl.ds(0, CED)], rows[b], isem[b]).wait()
            pltpu.make_async_copy(att_hbm.at[pl.ds(0, CED)], av[b], isem[b]).wait()

        _issue(wid, 0)

        def step(k, b):
            nb = 1 - b
            ck = wid + k * NW
            cn = wid + (k + 1) * NW

            @pl.when(cn < NCHD)
            def _():
                _issue(cn, nb)

            @pl.when(ck < NCHD)
            def _():
                _drain(b)
                pltpu.sync_copy(rows[b], acc_sh.at[idx[b]], add=True)
                pltpu.sync_copy(av[b], asum_sh.at[idx[b]], add=True)

        def pairk(k2, carry):
            for b in (0, 1):
                step(k2 * 2 + b, b)
            return carry

        lax.fori_loop(0, KMAXD // 2, pairk, 0)
        if KMAXD % 2:
            step(KMAXD - 1, 0)
        plsc.subcore_barrier()

        for j in range(RPT // RZB):
            rr = sid * RPT + j * RZB
            pltpu.sync_copy(acc_sh.at[pl.ds(rr, RZB)], acc_out.at[cid, pl.ds(rr, RZB)])
        pltpu.sync_copy(asum_sh.at[pl.ds(sid * RPT, RPT)],
                        asum_out.at[cid, pl.ds(sid * RPT, RPT)])

    return body


def _scatter(wm, att1, src, h):
    mesh = plsc.VectorSubcoreMesh(
        core_axis_name="c", subcore_axis_name="s", num_cores=NC, num_subcores=NS)
    f = functools.partial(
        pl.kernel,
        out_type=[
            jax.ShapeDtypeStruct((NC, NP, D), jnp.float32),
            jax.ShapeDtypeStruct((NC, NP), jnp.float32),
        ],
        mesh=mesh,
        scratch_types=[
            pltpu.VMEM_SHARED((NP, D), jnp.float32),
            pltpu.VMEM_SHARED((NP,), jnp.float32),
            pltpu.VMEM((CED,), jnp.int32),
            pltpu.VMEM((CED,), jnp.int32),
            pltpu.VMEM((CED, D), jnp.float32),
            pltpu.VMEM((CED, D), jnp.float32),
            pltpu.VMEM((CED,), jnp.float32),
            pltpu.VMEM((CED,), jnp.float32),
            pltpu.VMEM((ZW, D), jnp.float32),
            pltpu.VMEM((RPT,), jnp.float32),
            pltpu.SemaphoreType.DMA,
            pltpu.SemaphoreType.DMA,
            pltpu.SemaphoreType.DMA,
        ],
        name=f"edge_scatter_h{h}",
    )(_make_scatter_body(h))
    return f(wm, att1, src)


def _node_post_body(hv_ref, acc1_ref, acc2_ref, acc3_ref, acc4_ref,
                    asum1_ref, asum2_ref, asum3_ref, asum4_ref,
                    win_ref, bin_ref, wout_ref,
                    bout_ref, g0_ref, bt0_ref, g1_ref, bt1_ref, out_ref):
    acc = (acc1_ref[0] + acc1_ref[1] + acc2_ref[0] + acc2_ref[1]
           + acc3_ref[0] + acc3_ref[1] + acc4_ref[0] + acc4_ref[1])
    asum_p = (asum1_ref[0] + asum1_ref[1] + asum2_ref[0] + asum2_ref[1]
              + asum3_ref[0] + asum3_ref[1] + asum4_ref[0] + asum4_ref[1])
    asum = _unpack(asum_p.reshape(1, BN // 128, 128), BN)
    dh = jnp.where(asum > 0, acc / jnp.where(asum > 0, asum, 1.0), 0.0) / SCALE
    x = hv_ref[...] + dh
    mu = jnp.mean(x, axis=-1, keepdims=True)
    xc = x - mu
    var = jnp.mean(xc * xc, axis=-1, keepdims=True)
    x = g0_ref[...] * xc * jax.lax.rsqrt(var + EPS) + bt0_ref[...]
    hh = jnp.maximum(jnp.dot(x, win_ref[...], preferred_element_type=jnp.float32) + bin_ref[...], 0.0)
    ffn = jnp.dot(hh, wout_ref[...], preferred_element_type=jnp.float32) + bout_ref[...]
    y = x + ffn
    mu2 = jnp.mean(y, axis=-1, keepdims=True)
    yc = y - mu2
    var2 = jnp.mean(yc * yc, axis=-1, keepdims=True)
    out_ref[...] = g1_ref[...] * yc * jax.lax.rsqrt(var2 + EPS) + bt1_ref[...]


def _node_post(h_Vp, accs, asums,
               W_in, binr, W_out, boutr, g0r, bt0r, g1r, bt1r):
    return pl.pallas_call(
        _node_post_body,
        grid=(NP // BN,),
        in_specs=[
            pl.BlockSpec((BN, D), lambda i: (i, 0)),
            pl.BlockSpec((NC, BN, D), lambda i: (0, i, 0)),
            pl.BlockSpec((NC, BN, D), lambda i: (0, i, 0)),
            pl.BlockSpec((NC, BN, D), lambda i: (0, i, 0)),
            pl.BlockSpec((NC, BN, D), lambda i: (0, i, 0)),
            pl.BlockSpec((NC, BN // 128, 128), lambda i: (0, i, 0)),
            pl.BlockSpec((NC, BN // 128, 128), lambda i: (0, i, 0)),
            pl.BlockSpec((NC, BN // 128, 128), lambda i: (0, i, 0)),
            pl.BlockSpec((NC, BN // 128, 128), lambda i: (0, i, 0)),
            pl.BlockSpec((D, DFF), lambda i: (0, 0)),
            pl.BlockSpec((1, DFF), lambda i: (0, 0)),
            pl.BlockSpec((DFF, D), lambda i: (0, 0)),
            pl.BlockSpec((1, D), lambda i: (0, 0)),
            pl.BlockSpec((1, D), lambda i: (0, 0)),
            pl.BlockSpec((1, D), lambda i: (0, 0)),
            pl.BlockSpec((1, D), lambda i: (0, 0)),
            pl.BlockSpec((1, D), lambda i: (0, 0)),
        ],
        out_specs=pl.BlockSpec((BN, D), lambda i: (i, 0)),
        out_shape=jax.ShapeDtypeStruct((NP, D), jnp.float32),
    )(h_Vp, *accs, *asums,
      W_in, binr, W_out, boutr, g0r, bt0r, g1r, bt1r)


def kernel(h_V, h_E, edge_idx, W1, b1, W2, b2, W3, b3, A,
           W_in, b_in, W_out, b_out, g0, bt0, g1, bt1):
    src = edge_idx[0]
    W1a, W1b = W1[:D], W1[D:]
    Aa, Ab = A[:D], A[D:]
    P, q = _node_pre(h_V, W1a, b1.reshape(1, D), Aa)
    q1 = q.reshape(N)
    b2r, b3r = b2.reshape(1, D), b3.reshape(1, D)

    G = [None] * NH
    qe = [None] * NH
    wm = [None] * NH
    att = [None] * NH
    acc = [None] * NH
    asum = [None] * NH
    for h in range(NH):
        G[h], qe[h] = _gather(P, q1, src, h)
    for h in range(NH):
        wm[h], attP3 = _edge_mlp(G[h], h_E, qe[h].reshape(NBE, RB, 128),
                                 W1b, W2, b2r, W3, b3r, Ab, h)
        att[h] = attP3.reshape(E2)
    for h in range(NH):
        acc[h], asum[h] = _scatter(wm[h], att[h], src, h)

    h_Vp = jnp.pad(h_V, ((0, NP - N), (0, 0)))
    out = _node_post(h_Vp, acc,
                     [a.reshape(NC, NP // 128, 128) for a in asum],
                     W_in, b_in.reshape(1, DFF), W_out, b_out.reshape(1, D),
                     g0.reshape(1, D), bt0.reshape(1, D),
                     g1.reshape(1, D), bt1.reshape(1, D))
    return out[:N]
